# baseline (speedup 1.0000x reference)
import functools

import jax
import jax.numpy as jnp
from jax import lax
from jax.experimental import pallas as pl
from jax.experimental.pallas import tpu as pltpu
from jax.experimental.pallas import tpu_sc as plsc

_N = 4096
_F = 128
_K = 16
_NSEG = 4
_SEG = 1024
_RB = 256
_NODE_B = 256
_EDGE_B = _NODE_B * _K

_SELU_SCALE = 1.0507009873554804934193349852946
_SELU_ALPHA = 1.6732632423543772848170429916717


def _selu(v):
    return _SELU_SCALE * jnp.where(v > 0, v, _SELU_ALPHA * (jnp.exp(v) - 1.0))


def _knn_body(seg0, q_ref, c_ref, idx_ref, dist_ref):
    s = seg0 + pl.program_id(0)
    acc = None
    for d in range(3):
        qd = q_ref[d, :]
        cd = c_ref[d, :]
        diff = qd[:, None] - cd[None, :]
        sq = diff * diff
        acc = sq if acc is None else acc + sq
    col = lax.broadcasted_iota(jnp.int32, (_RB, _SEG), 1)
    base = s * _SEG
    big = jnp.int32(2**30)
    for t in range(_K + 1):
        m = jnp.min(acc, axis=1)
        cand = jnp.where(acc == m[:, None], col, big)
        j = jnp.min(cand, axis=1)
        idx_ref[t, :] = j + base
        dist_ref[t, :] = m
        acc = jnp.where(col == j[:, None], jnp.float32(jnp.inf), acc)


def _knn(coords_t, seg0, nseg):
    grid = (nseg, _SEG // _RB)
    rbs = _SEG // _RB
    return pl.pallas_call(
        functools.partial(_knn_body, seg0),
        grid=grid,
        in_specs=[
            pl.BlockSpec((3, _RB), lambda s, r: (0, (seg0 + s) * rbs + r)),
            pl.BlockSpec((3, _SEG), lambda s, r: (0, seg0 + s)),
        ],
        out_specs=[
            pl.BlockSpec((_K + 1, _RB), lambda s, r: (0, s * rbs + r)),
            pl.BlockSpec((_K + 1, _RB), lambda s, r: (0, s * rbs + r)),
        ],
        out_shape=[
            jax.ShapeDtypeStruct((_K + 1, nseg * _SEG), jnp.int32),
            jax.ShapeDtypeStruct((_K + 1, nseg * _SEG), jnp.float32),
        ],
    )(coords_t, coords_t)


def _proj_body(x_ref, w_ref, o_ref):
    o_ref[...] = jnp.dot(x_ref[...], w_ref[...],
                         preferred_element_type=jnp.float32)


def _proj(x, w):
    return pl.pallas_call(
        _proj_body,
        out_shape=jax.ShapeDtypeStruct((_N, _F), jnp.float32),
    )(x, w)


def _sc_gather(p, coords, idx_e):
    info = plsc.get_sparse_core_info()
    nc, ns = info.num_cores, info.num_subcores
    nw = nc * ns
    e = idx_e.shape[0]
    per_w = e // nw
    ch = 128
    n_ch = per_w // ch
    mesh = plsc.VectorSubcoreMesh(core_axis_name="c", subcore_axis_name="s")
    f32 = jnp.float32

    @functools.partial(
        pl.kernel,
        mesh=mesh,
        out_type=(
            jax.ShapeDtypeStruct((e, _F), f32),
            jax.ShapeDtypeStruct((e, _F), f32),
        ),
        scratch_types=[
            pltpu.VMEM((ch,), jnp.int32),
            pltpu.VMEM((ch, _F), f32),
            pltpu.VMEM((ch, _F), f32),
            pltpu.SemaphoreType.DMA,
        ],
    )
    def k(p_hbm, c_hbm, idx_hbm, g_out, cg_out,
          idx_v, rows_v, crows_v, sem):
        wid = lax.axis_index("s") * nc + lax.axis_index("c")
        wbase = wid * per_w

        def body(i, carry):
            base = wbase + i * ch
            pltpu.sync_copy(idx_hbm.at[pl.ds(base, ch)], idx_v)
            cp = pltpu.async_copy(p_hbm.at[idx_v], rows_v, sem)
            cc = pltpu.async_copy(c_hbm.at[idx_v], crows_v, sem)
            cp.wait()
            pltpu.sync_copy(rows_v, g_out.at[pl.ds(base, ch)])
            cc.wait()
            pltpu.sync_copy(crows_v, cg_out.at[pl.ds(base, ch)])
            return carry

        lax.fori_loop(0, n_ch, body, 0)

    return k(p, coords, idx_e)


def _edge_body(dsq_ref, g_ref, cg_ref, x_ref, cp_ref,
               w1row_ref, w1n_ref, b1_ref, w2_ref, b2_ref,
               wc_ref, bc_ref, wc2_ref, bc2_ref,
               wnt_ref, wnb_ref, bn_ref, wn2_ref, bn2_ref,
               h_ref, co_ref):
    x_b = x_ref[...]
    a = jnp.dot(x_b, w1n_ref[...], preferred_element_type=jnp.float32)
    a = a + b1_ref[...]
    dist3 = dsq_ref[...][:, :, None]
    w1row3 = w1row_ref[...][None, :, :]
    e1_3 = dist3 * w1row3 + a[:, None, :]
    e1 = e1_3.reshape(_EDGE_B, _F) + g_ref[...]
    e1 = _selu(e1)
    e2 = _selu(jnp.dot(e1, w2_ref[...],
                       preferred_element_type=jnp.float32) + b2_ref[...])
    c1 = _selu(jnp.dot(e2, wc_ref[...],
                       preferred_element_type=jnp.float32) + bc_ref[...])
    c1_3 = c1.reshape(_NODE_B, _K, _F)
    wc2_3 = wc2_ref[...][None, :, :]
    c2 = jnp.sum(c1_3 * wc2_3, axis=2) + bc2_ref[0, 0]
    cg3 = cg_ref[...]
    wsum = jnp.sum(c2, axis=1, keepdims=True)
    wcg = jnp.sum(c2[:, :, None] * cg3, axis=1)
    cp = cp_ref[...]
    trans = (cp * wsum - wcg) * (1.0 / _K)
    co_ref[...] = cp + trans
    ef = jnp.sum(e2.reshape(_NODE_B, _K, _F), axis=1)
    h1 = _selu(jnp.dot(ef, wnt_ref[...], preferred_element_type=jnp.float32)
               + jnp.dot(x_b, wnb_ref[...], preferred_element_type=jnp.float32)
               + bn_ref[...])
    h_ref[...] = jnp.dot(h1, wn2_ref[...],
                         preferred_element_type=jnp.float32) + bn2_ref[...]


def _edge(dsq, g, cg, x, cpad, w1row, w1n, b1, w2, b2, wc, bc, wc2, bc2,
          wnt, wnb, bn, wn2, bn2):
    n = dsq.shape[0]
    nb = n // _NODE_B
    full = lambda shape: pl.BlockSpec(shape, lambda i: (0, 0))
    return pl.pallas_call(
        _edge_body,
        grid=(nb,),
        in_specs=[
            pl.BlockSpec((_NODE_B, _K), lambda i: (i, 0)),
            pl.BlockSpec((_EDGE_B, _F), lambda i: (i, 0)),
            pl.BlockSpec((_NODE_B, _K, 3), lambda i: (i, 0, 0)),
            pl.BlockSpec((_NODE_B, _F), lambda i: (i, 0)),
            pl.BlockSpec((_NODE_B, 3), lambda i: (i, 0)),
            full((1, _F)),
            full((_F, _F)),
            full((1, _F)),
            full((_F, _F)),
            full((1, _F)),
            full((_F, _F)),
            full((1, _F)),
            full((1, _F)),
            full((1, 1)),
            full((_F, _F)),
            full((_F, _F)),
            full((1, _F)),
            full((_F, _F)),
            full((1, _F)),
        ],
        out_specs=[
            pl.BlockSpec((_NODE_B, _F), lambda i: (i, 0)),
            pl.BlockSpec((_NODE_B, 3), lambda i: (i, 0)),
        ],
        out_shape=[
            jax.ShapeDtypeStruct((n, _F), jnp.float32),
            jax.ShapeDtypeStruct((n, 3), jnp.float32),
        ],
    )(dsq, g, cg, x, cpad, w1row, w1n, b1, w2, b2, wc, bc, wc2, bc2,
      wnt, wnb, bn, wn2, bn2)


def kernel(x, coordinates, row_splits, W1, b1, W2, b2, Wc, bc, Wc2, bc2,
           Wn, bn, Wn2, bn2):
    del row_splits
    coords_t = coordinates.T
    p = _proj(x, W1[1 + _F:, :])
    cpad = jnp.pad(coordinates, ((0, 0), (0, _F - 3)))
    weights = (
        W1[0:1, :], W1[1:1 + _F, :], b1.reshape(1, _F),
        W2, b2.reshape(1, _F), Wc, bc.reshape(1, _F),
        Wc2.T, bc2.reshape(1, 1),
        Wn[:_F, :], Wn[_F:, :], bn.reshape(1, _F),
        Wn2, bn2.reshape(1, _F),
    )

    splits = 4
    nh = _NSEG // splits
    hn = nh * _SEG
    nbr, dsq, h, co = [], [], [], []
    for half in range(splits):
        lo = half * hn
        idx_t, dist_t = _knn(coords_t, half * nh, nh)
        nbr_h = idx_t.T[:, 1:]
        dsq_h = dist_t.T[:, 1:]
        g, cgrows = _sc_gather(p, cpad, nbr_h.reshape(-1))
        cg = cgrows[:, :3].reshape(hn, _K, 3)
        h_h, co_h = _edge(dsq_h, g, cg, x[lo:lo + hn],
                          coordinates[lo:lo + hn], *weights)
        nbr.append(nbr_h)
        dsq.append(dsq_h)
        h.append(h_h)
        co.append(co_h)
    return (jnp.concatenate(h), jnp.concatenate(co),
            jnp.concatenate(nbr), jnp.concatenate(dsq))


# speedup vs baseline: 1.0777x; 1.0777x over previous
import functools

import jax
import jax.numpy as jnp
from jax import lax
from jax.experimental import pallas as pl
from jax.experimental.pallas import tpu as pltpu
from jax.experimental.pallas import tpu_sc as plsc

_N = 4096
_F = 128
_K = 16
_NSEG = 4
_SEG = 1024
_RB = 256
_NODE_B = 256
_EDGE_B = _NODE_B * _K

_SELU_SCALE = 1.0507009873554804934193349852946
_SELU_ALPHA = 1.6732632423543772848170429916717


def _selu(v):
    return _SELU_SCALE * jnp.where(v > 0, v, _SELU_ALPHA * (jnp.exp(v) - 1.0))


def _knn_body(seg0, q_ref, c_ref, idx_ref, dist_ref):
    s = seg0 + pl.program_id(0)
    r = pl.program_id(1)
    acc = None
    for d in range(3):
        qd = q_ref[d, :]
        cd = c_ref[d, :]
        diff = qd[:, None] - cd[None, :]
        sq = diff * diff
        acc = sq if acc is None else acc + sq
    col = lax.broadcasted_iota(jnp.int32, (_RB, _SEG), 1)
    qcol = lax.broadcasted_iota(jnp.int32, (_RB, 1), 0) + r * _RB
    inf = jnp.float32(jnp.inf)
    acc = jnp.where(col == qcol, inf, acc)
    base = s * _SEG
    big = jnp.int32(2**30)
    for t in range(_K):
        m = jnp.min(acc, axis=1)
        cand = jnp.where(acc == m[:, None], col, big)
        j = jnp.min(cand, axis=1)
        idx_ref[t, :] = j + base
        dist_ref[t, :] = m
        if t != _K - 1:
            acc = jnp.where(col == j[:, None], inf, acc)


def _knn(coords_t, seg0, nseg):
    grid = (nseg, _SEG // _RB)
    rbs = _SEG // _RB
    return pl.pallas_call(
        functools.partial(_knn_body, seg0),
        grid=grid,
        in_specs=[
            pl.BlockSpec((3, _RB), lambda s, r: (0, (seg0 + s) * rbs + r)),
            pl.BlockSpec((3, _SEG), lambda s, r: (0, seg0 + s)),
        ],
        out_specs=[
            pl.BlockSpec((_K, _RB), lambda s, r: (0, s * rbs + r)),
            pl.BlockSpec((_K, _RB), lambda s, r: (0, s * rbs + r)),
        ],
        out_shape=[
            jax.ShapeDtypeStruct((_K, nseg * _SEG), jnp.int32),
            jax.ShapeDtypeStruct((_K, nseg * _SEG), jnp.float32),
        ],
    )(coords_t, coords_t)


def _proj_body(x_ref, w_ref, o_ref):
    o_ref[...] = jnp.dot(x_ref[...], w_ref[...],
                         preferred_element_type=jnp.float32)


def _proj(x, w):
    return pl.pallas_call(
        _proj_body,
        out_shape=jax.ShapeDtypeStruct((_N, _F), jnp.float32),
    )(x, w)


def _sc_gather(p, coords, idx_e):
    info = plsc.get_sparse_core_info()
    nc, ns = info.num_cores, info.num_subcores
    nw = nc * ns
    e = idx_e.shape[0]
    per_w = e // nw
    ch = 128
    n_ch = per_w // ch
    mesh = plsc.VectorSubcoreMesh(core_axis_name="c", subcore_axis_name="s")
    f32 = jnp.float32

    @functools.partial(
        pl.kernel,
        mesh=mesh,
        out_type=(
            jax.ShapeDtypeStruct((e, _F), f32),
            jax.ShapeDtypeStruct((e, _F), f32),
        ),
        scratch_types=[
            pltpu.VMEM((ch,), jnp.int32),
            pltpu.VMEM((ch, _F), f32),
            pltpu.VMEM((ch, _F), f32),
            pltpu.SemaphoreType.DMA,
        ],
    )
    def k(p_hbm, c_hbm, idx_hbm, g_out, cg_out,
          idx_v, rows_v, crows_v, sem):
        wid = lax.axis_index("s") * nc + lax.axis_index("c")
        wbase = wid * per_w

        def body(i, carry):
            base = wbase + i * ch
            pltpu.sync_copy(idx_hbm.at[pl.ds(base, ch)], idx_v)
            cp = pltpu.async_copy(p_hbm.at[idx_v], rows_v, sem)
            cc = pltpu.async_copy(c_hbm.at[idx_v], crows_v, sem)
            cp.wait()
            pltpu.sync_copy(rows_v, g_out.at[pl.ds(base, ch)])
            cc.wait()
            pltpu.sync_copy(crows_v, cg_out.at[pl.ds(base, ch)])
            return carry

        lax.fori_loop(0, n_ch, body, 0)

    return k(p, coords, idx_e)


def _edge_body(dsq_ref, g_ref, cg_ref, x_ref, cp_ref,
               w1row_ref, w1n_ref, b1_ref, w2_ref, b2_ref,
               wc_ref, bc_ref, wc2_ref, bc2_ref,
               wnt_ref, wnb_ref, bn_ref, wn2_ref, bn2_ref,
               h_ref, co_ref):
    x_b = x_ref[...]
    a = jnp.dot(x_b, w1n_ref[...], preferred_element_type=jnp.float32)
    a = a + b1_ref[...]
    dist3 = dsq_ref[...][:, :, None]
    w1row3 = w1row_ref[...][None, :, :]
    e1_3 = dist3 * w1row3 + a[:, None, :]
    e1 = e1_3.reshape(_EDGE_B, _F) + g_ref[...]
    e1 = _selu(e1)
    e2 = _selu(jnp.dot(e1, w2_ref[...],
                       preferred_element_type=jnp.float32) + b2_ref[...])
    c1 = _selu(jnp.dot(e2, wc_ref[...],
                       preferred_element_type=jnp.float32) + bc_ref[...])
    c1_3 = c1.reshape(_NODE_B, _K, _F)
    wc2_3 = wc2_ref[...][None, :, :]
    c2 = jnp.sum(c1_3 * wc2_3, axis=2) + bc2_ref[0, 0]
    cg3 = cg_ref[...]
    wsum = jnp.sum(c2, axis=1, keepdims=True)
    wcg = jnp.sum(c2[:, :, None] * cg3, axis=1)
    cp = cp_ref[...]
    trans = (cp * wsum - wcg) * (1.0 / _K)
    co_ref[...] = cp + trans
    ef = jnp.sum(e2.reshape(_NODE_B, _K, _F), axis=1)
    h1 = _selu(jnp.dot(ef, wnt_ref[...], preferred_element_type=jnp.float32)
               + jnp.dot(x_b, wnb_ref[...], preferred_element_type=jnp.float32)
               + bn_ref[...])
    h_ref[...] = jnp.dot(h1, wn2_ref[...],
                         preferred_element_type=jnp.float32) + bn2_ref[...]


def _edge(dsq, g, cg, x, cpad, w1row, w1n, b1, w2, b2, wc, bc, wc2, bc2,
          wnt, wnb, bn, wn2, bn2):
    n = dsq.shape[0]
    nb = n // _NODE_B
    full = lambda shape: pl.BlockSpec(shape, lambda i: (0, 0))
    return pl.pallas_call(
        _edge_body,
        grid=(nb,),
        in_specs=[
            pl.BlockSpec((_NODE_B, _K), lambda i: (i, 0)),
            pl.BlockSpec((_EDGE_B, _F), lambda i: (i, 0)),
            pl.BlockSpec((_NODE_B, _K, 3), lambda i: (i, 0, 0)),
            pl.BlockSpec((_NODE_B, _F), lambda i: (i, 0)),
            pl.BlockSpec((_NODE_B, 3), lambda i: (i, 0)),
            full((1, _F)),
            full((_F, _F)),
            full((1, _F)),
            full((_F, _F)),
            full((1, _F)),
            full((_F, _F)),
            full((1, _F)),
            full((1, _F)),
            full((1, 1)),
            full((_F, _F)),
            full((_F, _F)),
            full((1, _F)),
            full((_F, _F)),
            full((1, _F)),
        ],
        out_specs=[
            pl.BlockSpec((_NODE_B, _F), lambda i: (i, 0)),
            pl.BlockSpec((_NODE_B, 3), lambda i: (i, 0)),
        ],
        out_shape=[
            jax.ShapeDtypeStruct((n, _F), jnp.float32),
            jax.ShapeDtypeStruct((n, 3), jnp.float32),
        ],
    )(dsq, g, cg, x, cpad, w1row, w1n, b1, w2, b2, wc, bc, wc2, bc2,
      wnt, wnb, bn, wn2, bn2)


def kernel(x, coordinates, row_splits, W1, b1, W2, b2, Wc, bc, Wc2, bc2,
           Wn, bn, Wn2, bn2):
    del row_splits
    coords_t = coordinates.T
    p = _proj(x, W1[1 + _F:, :])
    cpad = jnp.pad(coordinates, ((0, 0), (0, _F - 3)))
    weights = (
        W1[0:1, :], W1[1:1 + _F, :], b1.reshape(1, _F),
        W2, b2.reshape(1, _F), Wc, bc.reshape(1, _F),
        Wc2.T, bc2.reshape(1, 1),
        Wn[:_F, :], Wn[_F:, :], bn.reshape(1, _F),
        Wn2, bn2.reshape(1, _F),
    )

    splits = 2
    nh = _NSEG // splits
    hn = nh * _SEG
    nbr, dsq, h, co = [], [], [], []
    for half in range(splits):
        lo = half * hn
        idx_t, dist_t = _knn(coords_t, half * nh, nh)
        nbr_h = idx_t.T
        dsq_h = dist_t.T
        g, cgrows = _sc_gather(p, cpad, nbr_h.reshape(-1))
        cg = cgrows[:, :3].reshape(hn, _K, 3)
        h_h, co_h = _edge(dsq_h, g, cg, x[lo:lo + hn],
                          coordinates[lo:lo + hn], *weights)
        nbr.append(nbr_h)
        dsq.append(dsq_h)
        h.append(h_h)
        co.append(co_h)
    return (jnp.concatenate(h), jnp.concatenate(co),
            jnp.concatenate(nbr), jnp.concatenate(dsq))


# speedup vs baseline: 1.0961x; 1.0171x over previous
import functools

import jax
import jax.numpy as jnp
from jax import lax
from jax.experimental import pallas as pl
from jax.experimental.pallas import tpu as pltpu
from jax.experimental.pallas import tpu_sc as plsc

_N = 4096
_F = 128
_K = 16
_NSEG = 4
_SEG = 1024
_RB = 256
_NODE_B = 256
_EDGE_B = _NODE_B * _K

_SELU_SCALE = 1.0507009873554804934193349852946
_SELU_ALPHA = 1.6732632423543772848170429916717


def _selu(v):
    return _SELU_SCALE * jnp.where(v > 0, v, _SELU_ALPHA * (jnp.exp(v) - 1.0))


def _knn_body(seg0, q_ref, c_ref, idx_ref, dist_ref):
    s = seg0 + pl.program_id(0)
    r = pl.program_id(1)
    acc = None
    for d in range(3):
        qd = q_ref[d, :]
        cd = c_ref[d, :]
        diff = qd[:, None] - cd[None, :]
        sq = diff * diff
        acc = sq if acc is None else acc + sq
    col = lax.broadcasted_iota(jnp.int32, (_RB, _SEG), 1)
    qcol = lax.broadcasted_iota(jnp.int32, (_RB, 1), 0) + r * _RB
    inf = jnp.float32(jnp.inf)
    acc = jnp.where(col == qcol, inf, acc)
    base = s * _SEG
    big = jnp.int32(2**30)
    for t in range(_K):
        m = jnp.min(acc, axis=1)
        cand = jnp.where(acc == m[:, None], col, big)
        j = jnp.min(cand, axis=1)
        idx_ref[t, :] = j + base
        dist_ref[t, :] = m
        if t != _K - 1:
            acc = jnp.where(col == j[:, None], inf, acc)


def _knn(coords_t, seg0, nseg):
    grid = (nseg, _SEG // _RB)
    rbs = _SEG // _RB
    return pl.pallas_call(
        functools.partial(_knn_body, seg0),
        grid=grid,
        in_specs=[
            pl.BlockSpec((3, _RB), lambda s, r: (0, (seg0 + s) * rbs + r)),
            pl.BlockSpec((3, _SEG), lambda s, r: (0, seg0 + s)),
        ],
        out_specs=[
            pl.BlockSpec((_K, _RB), lambda s, r: (0, s * rbs + r)),
            pl.BlockSpec((_K, _RB), lambda s, r: (0, s * rbs + r)),
        ],
        out_shape=[
            jax.ShapeDtypeStruct((_K, nseg * _SEG), jnp.int32),
            jax.ShapeDtypeStruct((_K, nseg * _SEG), jnp.float32),
        ],
    )(coords_t, coords_t)


def _proj_body(x_ref, w_ref, o_ref):
    o_ref[...] = jnp.dot(x_ref[...], w_ref[...],
                         preferred_element_type=jnp.float32)


def _proj(x, w):
    return pl.pallas_call(
        _proj_body,
        out_shape=jax.ShapeDtypeStruct((_N, _F), jnp.float32),
    )(x, w)


def _sc_gather(p, coords, idx_e):
    info = plsc.get_sparse_core_info()
    nc, ns = info.num_cores, info.num_subcores
    nw = nc * ns
    e = idx_e.shape[0]
    per_w = e // nw
    ch = 128
    n_ch = per_w // ch
    mesh = plsc.VectorSubcoreMesh(core_axis_name="c", subcore_axis_name="s")
    f32 = jnp.float32

    @functools.partial(
        pl.kernel,
        mesh=mesh,
        out_type=(
            jax.ShapeDtypeStruct((e, _F), f32),
            jax.ShapeDtypeStruct((e, _F), f32),
        ),
        scratch_types=[
            pltpu.VMEM((ch,), jnp.int32),
            pltpu.VMEM((ch, _F), f32),
            pltpu.VMEM((ch, _F), f32),
            pltpu.SemaphoreType.DMA,
        ],
    )
    def k(p_hbm, c_hbm, idx_hbm, g_out, cg_out,
          idx_v, rows_v, crows_v, sem):
        wid = lax.axis_index("s") * nc + lax.axis_index("c")
        wbase = wid * per_w

        def body(i, carry):
            base = wbase + i * ch
            pltpu.sync_copy(idx_hbm.at[pl.ds(base, ch)], idx_v)
            cp = pltpu.async_copy(p_hbm.at[idx_v], rows_v, sem)
            cc = pltpu.async_copy(c_hbm.at[idx_v], crows_v, sem)
            cp.wait()
            pltpu.sync_copy(rows_v, g_out.at[pl.ds(base, ch)])
            cc.wait()
            pltpu.sync_copy(crows_v, cg_out.at[pl.ds(base, ch)])
            return carry

        lax.fori_loop(0, n_ch, body, 0)

    return k(p, coords, idx_e)


def _edge_body(dsq_ref, g_ref, cg_ref, x_ref, cp_ref,
               w1row_ref, w1n_ref, b1_ref, w2_ref, b2_ref,
               wc_ref, bc_ref, wc2_ref, bc2_ref,
               wnt_ref, wnb_ref, bn_ref, wn2_ref, bn2_ref,
               h_ref, co_ref):
    x_b = x_ref[...]
    a = jnp.dot(x_b, w1n_ref[...], preferred_element_type=jnp.float32)
    a = a + b1_ref[...]
    dist3 = dsq_ref[...][:, :, None]
    w1row3 = w1row_ref[...][None, :, :]
    e1_3 = dist3 * w1row3 + a[:, None, :]
    e1 = e1_3.reshape(_EDGE_B, _F) + g_ref[...]
    e1 = _selu(e1)
    e2 = _selu(jnp.dot(e1, w2_ref[...],
                       preferred_element_type=jnp.float32) + b2_ref[...])
    c1 = _selu(jnp.dot(e2, wc_ref[...],
                       preferred_element_type=jnp.float32) + bc_ref[...])
    c1_3 = c1.reshape(_NODE_B, _K, _F)
    wc2_3 = wc2_ref[...][None, :, :]
    c2 = jnp.sum(c1_3 * wc2_3, axis=2) + bc2_ref[0, 0]
    cg3 = cg_ref[...]
    wsum = jnp.sum(c2, axis=1, keepdims=True)
    wcg = jnp.sum(c2[:, :, None] * cg3, axis=1)
    cp = cp_ref[...]
    trans = (cp * wsum - wcg) * (1.0 / _K)
    co_ref[...] = cp + trans
    ef = jnp.sum(e2.reshape(_NODE_B, _K, _F), axis=1)
    h1 = _selu(jnp.dot(ef, wnt_ref[...], preferred_element_type=jnp.float32)
               + jnp.dot(x_b, wnb_ref[...], preferred_element_type=jnp.float32)
               + bn_ref[...])
    h_ref[...] = jnp.dot(h1, wn2_ref[...],
                         preferred_element_type=jnp.float32) + bn2_ref[...]


def _edge(dsq, g, cg, x, cpad, node_off, w1row, w1n, b1, w2, b2, wc, bc,
          wc2, bc2, wnt, wnb, bn, wn2, bn2):
    n = dsq.shape[0]
    nb = n // _NODE_B
    ob = node_off // _NODE_B
    full = lambda shape: pl.BlockSpec(shape, lambda i: (0, 0))
    return pl.pallas_call(
        _edge_body,
        grid=(nb,),
        in_specs=[
            pl.BlockSpec((_NODE_B, _K), lambda i: (i, 0)),
            pl.BlockSpec((_EDGE_B, _F), lambda i: (i, 0)),
            pl.BlockSpec((_NODE_B, _K, 3), lambda i: (i, 0, 0)),
            pl.BlockSpec((_NODE_B, _F), lambda i: (ob + i, 0)),
            pl.BlockSpec((_NODE_B, 3), lambda i: (ob + i, 0)),
            full((1, _F)),
            full((_F, _F)),
            full((1, _F)),
            full((_F, _F)),
            full((1, _F)),
            full((_F, _F)),
            full((1, _F)),
            full((1, _F)),
            full((1, 1)),
            full((_F, _F)),
            full((_F, _F)),
            full((1, _F)),
            full((_F, _F)),
            full((1, _F)),
        ],
        out_specs=[
            pl.BlockSpec((_NODE_B, _F), lambda i: (i, 0)),
            pl.BlockSpec((_NODE_B, 3), lambda i: (i, 0)),
        ],
        out_shape=[
            jax.ShapeDtypeStruct((n, _F), jnp.float32),
            jax.ShapeDtypeStruct((n, 3), jnp.float32),
        ],
    )(dsq, g, cg, x, cpad, w1row, w1n, b1, w2, b2, wc, bc, wc2, bc2,
      wnt, wnb, bn, wn2, bn2)


def kernel(x, coordinates, row_splits, W1, b1, W2, b2, Wc, bc, Wc2, bc2,
           Wn, bn, Wn2, bn2):
    del row_splits
    coords_t = coordinates.T
    p = _proj(x, W1[1 + _F:, :])
    cpad = jnp.pad(coordinates, ((0, 0), (0, _F - 3)))
    weights = (
        W1[0:1, :], W1[1:1 + _F, :], b1.reshape(1, _F),
        W2, b2.reshape(1, _F), Wc, bc.reshape(1, _F),
        Wc2.T, bc2.reshape(1, 1),
        Wn[:_F, :], Wn[_F:, :], bn.reshape(1, _F),
        Wn2, bn2.reshape(1, _F),
    )

    splits = 2
    nh = _NSEG // splits
    hn = nh * _SEG
    nbr, dsq, h, co = [], [], [], []
    for half in range(splits):
        lo = half * hn
        idx_t, dist_t = _knn(coords_t, half * nh, nh)
        nbr_h = idx_t.T
        dsq_h = dist_t.T
        g, cgrows = _sc_gather(p, cpad, nbr_h.reshape(-1))
        cg = cgrows[:, :3].reshape(hn, _K, 3)
        h_h, co_h = _edge(dsq_h, g, cg, x, coordinates, lo, *weights)
        nbr.append(nbr_h)
        dsq.append(dsq_h)
        h.append(h_h)
        co.append(co_h)
    return (jnp.concatenate(h), jnp.concatenate(co),
            jnp.concatenate(nbr), jnp.concatenate(dsq))
